# baseline (speedup 1.0000x reference)
import functools

import jax
import jax.numpy as jnp
from jax import lax
from jax.experimental import pallas as pl
from jax.experimental.pallas import tpu as pltpu
from jax.experimental.pallas import tpu_sc as plsc

D_MODEL = 64
NUM_WORKERS = 32
TOTAL = 1024 * 200
B_PER_W = TOTAL // NUM_WORKERS
CHUNK = 800
NCHUNK = B_PER_W // CHUNK


@functools.partial(
    pl.kernel,
    out_type=jax.ShapeDtypeStruct((TOTAL, D_MODEL), jnp.float32),
    mesh=plsc.VectorSubcoreMesh(core_axis_name="c", subcore_axis_name="s"),
    compiler_params=pltpu.CompilerParams(use_tc_tiling_on_sc=False),
    scratch_types=[
        pltpu.VMEM((B_PER_W,), jnp.int32),
        pltpu.VMEM((CHUNK, D_MODEL), jnp.float32),
        pltpu.SemaphoreType.DMA,
    ],
)
def _embed_gather(ids_hbm, table_hbm, out_hbm, idx_v, rows_v, sem):
    wid = lax.axis_index("s") * 2 + lax.axis_index("c")
    base = wid * B_PER_W
    pltpu.sync_copy(ids_hbm.at[pl.ds(base, B_PER_W)], idx_v)

    def body(g, carry):
        off = g * CHUNK
        pltpu.async_copy(
            table_hbm.at[idx_v.at[pl.ds(off, CHUNK)]], rows_v, sem
        ).wait()
        pltpu.sync_copy(rows_v, out_hbm.at[pl.ds(base + off, CHUNK)])
        return carry

    lax.fori_loop(0, NCHUNK, body, 0)


def kernel(ids, emb_weight):
    flat = ids.reshape(-1)
    out = _embed_gather(flat, emb_weight)
    return out.reshape(ids.shape[0], ids.shape[1], D_MODEL)


# speedup vs baseline: 1.0055x; 1.0055x over previous
import functools

import jax
import jax.numpy as jnp
from jax import lax
from jax.experimental import pallas as pl
from jax.experimental.pallas import tpu as pltpu
from jax.experimental.pallas import tpu_sc as plsc

D_MODEL = 64
NUM_WORKERS = 32
TOTAL = 1024 * 200
B_PER_W = TOTAL // NUM_WORKERS
CHUNK = 800
NCHUNK = B_PER_W // CHUNK


@functools.partial(
    pl.kernel,
    out_type=jax.ShapeDtypeStruct((TOTAL, D_MODEL), jnp.float32),
    mesh=plsc.VectorSubcoreMesh(core_axis_name="c", subcore_axis_name="s"),
    compiler_params=pltpu.CompilerParams(use_tc_tiling_on_sc=False),
    scratch_types=[
        pltpu.VMEM((B_PER_W,), jnp.int32),
        pltpu.VMEM((CHUNK, D_MODEL), jnp.float32),
        pltpu.VMEM((CHUNK, D_MODEL), jnp.float32),
        pltpu.SemaphoreType.DMA,
        pltpu.SemaphoreType.DMA,
        pltpu.SemaphoreType.DMA,
        pltpu.SemaphoreType.DMA,
    ],
)
def _embed_gather(ids_hbm, table_hbm, out_hbm, idx_v, rows0, rows1,
                  gsem0, gsem1, osem0, osem1):
    wid = lax.axis_index("s") * 2 + lax.axis_index("c")
    base = wid * B_PER_W
    pltpu.sync_copy(ids_hbm.at[pl.ds(base, B_PER_W)], idx_v)

    bufs = (rows0, rows1)
    gsems = (gsem0, gsem1)
    osems = (osem0, osem1)

    def start_gather(g):
        return pltpu.async_copy(
            table_hbm.at[idx_v.at[pl.ds(g * CHUNK, CHUNK)]],
            bufs[g % 2], gsems[g % 2])

    def start_write(g):
        return pltpu.async_copy(
            bufs[g % 2], out_hbm.at[pl.ds(base + g * CHUNK, CHUNK)],
            osems[g % 2])

    gathers = [start_gather(0), start_gather(1)]
    writes = [None, None]
    for g in range(NCHUNK):
        gathers[g % 2].wait()
        w = start_write(g)
        writes[g % 2] = w
        nxt = g + 2
        if nxt < NCHUNK:
            writes[nxt % 2].wait()
            gathers[nxt % 2] = start_gather(nxt)
    writes[(NCHUNK - 2) % 2].wait()
    writes[(NCHUNK - 1) % 2].wait()


def kernel(ids, emb_weight):
    flat = ids.reshape(-1)
    out = _embed_gather(flat, emb_weight)
    return out.reshape(ids.shape[0], ids.shape[1], D_MODEL)
